# Initial kernel scaffold; baseline (speedup 1.0000x reference)
import jax
import jax.numpy as jnp
from jax.experimental import pallas as pl


def kernel(story, query, word_embed, pos_embed):
    raise NotImplementedError("write your pallas kernel here")


# speedup vs baseline: 10.8931x; 10.8931x over previous
import functools

import jax
import jax.numpy as jnp
from jax import lax
from jax.experimental import pallas as pl
from jax.experimental.pallas import tpu as pltpu
from jax.experimental.pallas import tpu_sc as plsc


def _sc_geometry():
    try:
        info = plsc.get_sparse_core_info()
        return info.num_cores, info.num_subcores
    except Exception:
        return 2, 16


@functools.partial(jax.jit, static_argnames=("nseg", "W", "E", "cs"))
def _bag_sum(idx, table, pos, nseg, W, E, cs):
    NC, NS = _sc_geometry()
    nw = NC * NS
    seg_per_tile = nseg // nw
    nchunk = seg_per_tile // cs
    mesh = plsc.VectorSubcoreMesh(core_axis_name="c", subcore_axis_name="s")

    @functools.partial(
        pl.kernel,
        mesh=mesh,
        compiler_params=pltpu.CompilerParams(use_tc_tiling_on_sc=False),
        out_type=jax.ShapeDtypeStruct((nseg, E), jnp.float32),
        scratch_types=[
            pltpu.VMEM((cs * W,), jnp.int32),
            pltpu.VMEM((cs * W, E), jnp.float32),
            pltpu.VMEM((32, E), jnp.float32),
            pltpu.VMEM((cs, E), jnp.float32),
            pltpu.SemaphoreType.DMA,
        ],
    )
    def body(idx_hbm, table_hbm, pos_hbm, out_hbm, idx_v, rows_v, pos_v, out_v, sem):
        wid = lax.axis_index("s") * NC + lax.axis_index("c")
        pltpu.sync_copy(pos_hbm, pos_v)
        seg0 = wid * seg_per_tile

        def chunk_body(ci, carry):
            base = seg0 + ci * cs
            pltpu.sync_copy(idx_hbm.at[pl.ds(base * W, cs * W)], idx_v)
            pltpu.async_copy(table_hbm.at[idx_v], rows_v, sem).wait()

            def seg_body(s, c2):
                for j in range(E // 16):
                    acc = jnp.zeros((16,), jnp.float32)
                    for w in range(W):
                        r = rows_v[s * W + w, pl.ds(j * 16, 16)]
                        p = pos_v[w, pl.ds(j * 16, 16)]
                        acc = acc + r * p
                    out_v[s, pl.ds(j * 16, 16)] = acc
                return c2

            lax.fori_loop(0, cs, seg_body, 0)
            pltpu.sync_copy(out_v, out_hbm.at[pl.ds(base, cs)])
            return carry

        lax.fori_loop(0, nchunk, chunk_body, 0)

    return body(idx, table, pos)


def kernel(story, query, word_embed, pos_embed):
    B, S, W = story.shape
    E = word_embed.shape[1]
    nseg = B * S + B
    idx = jnp.concatenate([story.reshape(-1), query.reshape(-1)])
    out = _bag_sum(idx, word_embed, pos_embed, nseg=nseg, W=W, E=E, cs=64)
    sentence_sum = out[: B * S].reshape(B, S, E)
    query_sum = out[B * S :]
    return (sentence_sum, query_sum)


# speedup vs baseline: 18.2070x; 1.6714x over previous
import functools

import jax
import jax.numpy as jnp
from jax import lax
from jax.experimental import pallas as pl
from jax.experimental.pallas import tpu as pltpu
from jax.experimental.pallas import tpu_sc as plsc


def _sc_geometry():
    try:
        info = plsc.get_sparse_core_info()
        return info.num_cores, info.num_subcores
    except Exception:
        return 2, 16


@functools.partial(jax.jit, static_argnames=("nseg", "W", "E", "cs"))
def _bag_sum(idx, table, pos, nseg, W, E, cs):
    NC, NS = _sc_geometry()
    nw = NC * NS
    seg_per_tile = nseg // nw
    nchunk = seg_per_tile // cs
    assert nchunk % 2 == 0
    mesh = plsc.VectorSubcoreMesh(core_axis_name="c", subcore_axis_name="s")

    @functools.partial(
        pl.kernel,
        mesh=mesh,
        compiler_params=pltpu.CompilerParams(use_tc_tiling_on_sc=False),
        out_type=jax.ShapeDtypeStruct((nseg, E), jnp.float32),
        scratch_types=[
            pltpu.VMEM((2, cs * W), jnp.int32),
            pltpu.VMEM((2, cs * W, E), jnp.float32),
            pltpu.VMEM((32, E), jnp.float32),
            pltpu.VMEM((2, cs, E), jnp.float32),
            pltpu.SemaphoreType.DMA,
            pltpu.SemaphoreType.DMA,
            pltpu.SemaphoreType.DMA,
            pltpu.SemaphoreType.DMA,
            pltpu.SemaphoreType.DMA,
            pltpu.SemaphoreType.DMA,
        ],
    )
    def body(idx_hbm, table_hbm, pos_hbm, out_hbm, idx_v, rows_v, pos_v, out_v,
             sem_i0, sem_i1, sem_g0, sem_g1, sem_o0, sem_o1):
        sem_i = (sem_i0, sem_i1)
        sem_g = (sem_g0, sem_g1)
        sem_o = (sem_o0, sem_o1)
        wid = lax.axis_index("s") * NC + lax.axis_index("c")
        pltpu.sync_copy(pos_hbm, pos_v)
        seg0 = wid * seg_per_tile

        def start_idx(i, b):
            pltpu.make_async_copy(
                idx_hbm.at[pl.ds((seg0 + i * cs) * W, cs * W)],
                idx_v.at[b], sem_i[b]).start()

        def start_gather(b):
            pltpu.make_async_copy(
                table_hbm.at[idx_v.at[b]], rows_v.at[b], sem_g[b]).start()

        def compute(b):
            rows = rows_v.at[b]
            out = out_v.at[b]
            for j in range(E // 16):
                sl = pl.ds(j * 16, 16)
                pvs = [pos_v[w, sl] for w in range(W)]

                def seg_body(s, carry, sl=sl, pvs=pvs, nacc=4):
                    base = s * W
                    accs = [rows[base + a, sl] * pvs[a] for a in range(nacc)]
                    for w in range(nacc, W):
                        a = w % nacc
                        accs[a] = accs[a] + rows[base + w, sl] * pvs[w]
                    out[s, sl] = (accs[0] + accs[1]) + (accs[2] + accs[3])
                    return carry

                lax.fori_loop(0, cs, seg_body, 0, unroll=2)

        start_idx(0, 0)
        pltpu.make_async_copy(
            idx_hbm.at[pl.ds(seg0 * W, cs * W)], idx_v.at[0], sem_i[0]).wait()
        start_gather(0)
        start_idx(1, 1)

        def pair_body(ci, carry):
            for b in range(2):
                i = 2 * ci + b
                pltpu.make_async_copy(
                    table_hbm.at[idx_v.at[b]], rows_v.at[b], sem_g[b]).wait()

                @pl.when(i + 2 < nchunk)
                def _():
                    start_idx(i + 2, b)

                @pl.when(i + 1 < nchunk)
                def _():
                    pltpu.make_async_copy(
                        idx_hbm.at[pl.ds((seg0 + (i + 1) * cs) * W, cs * W)],
                        idx_v.at[1 - b], sem_i[1 - b]).wait()
                    start_gather(1 - b)

                @pl.when(i >= 2)
                def _():
                    pltpu.make_async_copy(
                        out_v.at[b], out_hbm.at[pl.ds(seg0 + (i - 2) * cs, cs)],
                        sem_o[b]).wait()

                compute(b)
                pltpu.make_async_copy(
                    out_v.at[b], out_hbm.at[pl.ds(seg0 + i * cs, cs)],
                    sem_o[b]).start()
            return carry

        lax.fori_loop(0, nchunk // 2, pair_body, 0)
        pltpu.make_async_copy(
            out_v.at[0], out_hbm.at[pl.ds(seg0 + (nchunk - 2) * cs, cs)],
            sem_o[0]).wait()
        pltpu.make_async_copy(
            out_v.at[1], out_hbm.at[pl.ds(seg0 + (nchunk - 1) * cs, cs)],
            sem_o[1]).wait()

    return body(idx, table, pos)


def kernel(story, query, word_embed, pos_embed):
    B, S, W = story.shape
    E = word_embed.shape[1]
    nseg = B * S + B
    idx = jnp.concatenate([story.reshape(-1), query.reshape(-1)])
    out = _bag_sum(idx, word_embed, pos_embed, nseg=nseg, W=W, E=E, cs=32)
    sentence_sum = out[: B * S].reshape(B, S, E)
    query_sum = out[B * S :]
    return (sentence_sum, query_sum)


# speedup vs baseline: 22.0448x; 1.2108x over previous
import functools

import jax
import jax.numpy as jnp
from jax import lax
from jax.experimental import pallas as pl
from jax.experimental.pallas import tpu as pltpu
from jax.experimental.pallas import tpu_sc as plsc


def _sc_geometry():
    try:
        info = plsc.get_sparse_core_info()
        return info.num_cores, info.num_subcores
    except Exception:
        return 2, 16


@functools.partial(jax.jit, static_argnames=("ns_story", "ns_query", "W", "E", "cs"))
def _bag_sum(story_idx, query_idx, table, pos, ns_story, ns_query, W, E, cs):
    NC, NS = _sc_geometry()
    nw = NC * NS
    nchunk_s = ns_story // (nw * cs)
    nchunk_q = ns_query // (nw * cs)
    mesh = plsc.VectorSubcoreMesh(core_axis_name="c", subcore_axis_name="s")

    @functools.partial(
        pl.kernel,
        mesh=mesh,
        compiler_params=pltpu.CompilerParams(use_tc_tiling_on_sc=False),
        out_type=(
            jax.ShapeDtypeStruct((ns_story, E), jnp.float32),
            jax.ShapeDtypeStruct((ns_query, E), jnp.float32),
        ),
        scratch_types=[
            pltpu.VMEM((2, cs * W), jnp.int32),
            pltpu.VMEM((2, cs * W, E), jnp.float32),
            pltpu.VMEM((32, E), jnp.float32),
            pltpu.VMEM((2, cs, E), jnp.float32),
            pltpu.SemaphoreType.DMA,
            pltpu.SemaphoreType.DMA,
            pltpu.SemaphoreType.DMA,
            pltpu.SemaphoreType.DMA,
            pltpu.SemaphoreType.DMA,
            pltpu.SemaphoreType.DMA,
        ],
    )
    def body(sidx_hbm, qidx_hbm, table_hbm, pos_hbm, out_s_hbm, out_q_hbm,
             idx_v, rows_v, pos_v, out_v,
             sem_i0, sem_i1, sem_g0, sem_g1, sem_o0, sem_o1):
        sem_i = (sem_i0, sem_i1)
        sem_g = (sem_g0, sem_g1)
        sem_o = (sem_o0, sem_o1)
        wid = lax.axis_index("s") * NC + lax.axis_index("c")
        pltpu.sync_copy(pos_hbm, pos_v)

        def compute(b):
            rows = rows_v.at[b]
            out = out_v.at[b]
            for j in range(E // 16):
                sl = pl.ds(j * 16, 16)
                pvs = [pos_v[w, sl] for w in range(W)]

                def seg_body(s, carry, sl=sl, pvs=pvs, nacc=4):
                    base = s * W
                    accs = [rows[base + a, sl] * pvs[a] for a in range(nacc)]
                    for w in range(nacc, W):
                        a = w % nacc
                        accs[a] = accs[a] + rows[base + w, sl] * pvs[w]
                    out[s, sl] = (accs[0] + accs[1]) + (accs[2] + accs[3])
                    return carry

                lax.fori_loop(0, cs, seg_body, 0, unroll=2)

        def run_pipeline(idx_hbm, out_hbm, nchunk):
            seg0 = wid * (nchunk * cs)

            def start_idx(i, b):
                pltpu.make_async_copy(
                    idx_hbm.at[pl.ds((seg0 + i * cs) * W, cs * W)],
                    idx_v.at[b], sem_i[b]).start()

            def wait_idx(i, b):
                pltpu.make_async_copy(
                    idx_hbm.at[pl.ds((seg0 + i * cs) * W, cs * W)],
                    idx_v.at[b], sem_i[b]).wait()

            def start_gather(b):
                pltpu.make_async_copy(
                    table_hbm.at[idx_v.at[b]], rows_v.at[b], sem_g[b]).start()

            def wait_gather(b):
                pltpu.make_async_copy(
                    table_hbm.at[idx_v.at[b]], rows_v.at[b], sem_g[b]).wait()

            def out_copy(i, b):
                return pltpu.make_async_copy(
                    out_v.at[b], out_hbm.at[pl.ds(seg0 + i * cs, cs)], sem_o[b])

            start_idx(0, 0)
            wait_idx(0, 0)
            start_gather(0)

            @pl.when(nchunk > 1)
            def _():
                start_idx(1, 1)

            def pair_body(ci, carry):
                for b in range(2):
                    i = 2 * ci + b
                    wait_gather(b)

                    @pl.when(i + 2 < nchunk)
                    def _():
                        start_idx(i + 2, b)

                    @pl.when(i + 1 < nchunk)
                    def _():
                        wait_idx(i + 1, 1 - b)
                        start_gather(1 - b)

                    @pl.when(i >= 2)
                    def _():
                        out_copy(i - 2, b).wait()

                    compute(b)
                    out_copy(i, b).start()
                return carry

            lax.fori_loop(0, nchunk // 2, pair_body, 0)
            out_copy(nchunk - 2, 0).wait()
            out_copy(nchunk - 1, 1).wait()

        run_pipeline(sidx_hbm, out_s_hbm, nchunk_s)
        run_pipeline(qidx_hbm, out_q_hbm, nchunk_q)

    return body(story_idx, query_idx, table, pos)


def kernel(story, query, word_embed, pos_embed):
    B, S, W = story.shape
    E = word_embed.shape[1]
    out_s, out_q = _bag_sum(
        story.reshape(-1), query.reshape(-1), word_embed, pos_embed,
        ns_story=B * S, ns_query=B, W=W, E=E, cs=32)
    return (out_s.reshape(B, S, E), out_q)
